# baseline (speedup 1.0000x reference)
import functools

import jax
import jax.numpy as jnp
from jax import lax
from jax.experimental import pallas as pl
from jax.experimental.pallas import tpu as pltpu
from jax.experimental.pallas import tpu_sc as plsc

NUM_TOKENS = 65536
D = 1024
B = 128
HW = 1024
NC = 2
NS = 16
NW = NC * NS
PW = HW // NW
LANES = 16
NBUF = 3
JPC = 4
CHUNK = JPC * PW
NTAIL = 17
POSV = PW * D - NTAIL * LANES

_mesh = plsc.VectorSubcoreMesh(core_axis_name="c", subcore_axis_name="s")


@functools.partial(
    pl.kernel,
    mesh=_mesh,
    out_type=jax.ShapeDtypeStruct((B, HW, D), jnp.float32),
    scratch_types=[
        pltpu.VMEM((2, CHUNK), jnp.int32),
        pltpu.VMEM((POSV,), jnp.float32),
        pltpu.VMEM((PW, D), jnp.float32),
        pltpu.VMEM((PW, D), jnp.float32),
        pltpu.VMEM((PW, D), jnp.float32),
        pltpu.SemaphoreType.DMA,
        pltpu.SemaphoreType.DMA,
        pltpu.SemaphoreType.DMA,
        pltpu.SemaphoreType.DMA,
        pltpu.SemaphoreType.DMA,
        pltpu.SemaphoreType.DMA,
        pltpu.SemaphoreType.DMA,
    ],
)
def _embed(idx_hbm, table_hbm, pos_hbm, pos2d_hbm, out_hbm, idxring, posv,
           r0, r1, r2, g0, g1, g2, s0, s1, s2, sidx):
    wid = lax.axis_index("s") * NC + lax.axis_index("c")
    bufs = (r0, r1, r2)
    gsems = (g0, g1, g2)
    ssems = (s0, s1, s2)

    pltpu.sync_copy(
        pos2d_hbm.at[pl.ds(wid * PW + PW - 8, 8), :],
        r0.at[pl.ds(0, 8), :],
    )
    tail_c0 = D // LANES - NTAIL
    tail = tuple(
        r0[7, pl.ds((tail_c0 + i) * LANES, LANES)] for i in range(NTAIL)
    )

    pltpu.sync_copy(pos_hbm.at[pl.ds(wid * PW * D, POSV)], posv)
    pltpu.sync_copy(idx_hbm.at[wid, pl.ds(0, CHUNK)], idxring.at[0])

    def issue_idx(c):
        pltpu.make_async_copy(
            idx_hbm.at[wid, pl.ds(c * CHUNK, CHUNK)],
            idxring.at[c % 2], sidx,
        ).start()

    def wait_idx():
        pltpu.make_async_copy(
            idx_hbm.at[wid, pl.ds(0, CHUNK)], idxring.at[0], sidx,
        ).wait()

    def start_gather(j, k):
        chunk_slot = (j // JPC) % 2
        off = (j % JPC) * PW
        pltpu.make_async_copy(
            table_hbm.at[idxring.at[chunk_slot, pl.ds(off, PW)]],
            bufs[k], gsems[k],
        ).start()

    def wait_gather(k):
        pltpu.make_async_copy(
            table_hbm.at[idxring.at[0, pl.ds(0, PW)]], bufs[k], gsems[k],
        ).wait()

    def start_scatter(b, k):
        pltpu.make_async_copy(
            bufs[k], out_hbm.at[b, pl.ds(wid * PW, PW), :], ssems[k]
        ).start()

    def wait_scatter(k):
        pltpu.make_async_copy(
            bufs[k], out_hbm.at[0, pl.ds(wid * PW, PW), :], ssems[k]
        ).wait()

    def add_pos(k):
        def add_row(r, carry):
            base = pl.multiple_of(r * D, D)
            for c in range(D // LANES):
                plsc.addupdate(
                    bufs[k].at[r, pl.ds(c * LANES, LANES)],
                    posv[pl.ds(base + c * LANES, LANES)],
                )
            return carry
        lax.fori_loop(0, PW - 1, add_row, 0)
        r31 = (PW - 1) * D
        for c in range(D // LANES - NTAIL):
            plsc.addupdate(
                bufs[k].at[PW - 1, pl.ds(c * LANES, LANES)],
                posv[pl.ds(r31 + c * LANES, LANES)],
            )
        for i in range(NTAIL):
            c = D // LANES - NTAIL + i
            plsc.addupdate(bufs[k].at[PW - 1, pl.ds(c * LANES, LANES)], tail[i])

    start_gather(0, 0)
    start_gather(1, 1)
    issue_idx(1)
    wait_gather(0); add_pos(0); start_scatter(0, 0); start_gather(2, 2)
    wait_gather(1); add_pos(1); start_scatter(1, 1)
    wait_scatter(0); start_gather(3, 0)

    def body(m, carry):
        for t in range(NBUF):
            b = 3 * m + 2 + t
            k = (2 + t) % NBUF
            kn = (k + 2) % NBUF
            wait_gather(k)

            @pl.when(jnp.logical_and(b % JPC == 0, b <= B - 2 * JPC))
            def _():
                issue_idx(b // JPC + 1)

            add_pos(k)
            start_scatter(b, k)
            wait_scatter(kn)

            @pl.when(jnp.logical_and(b % JPC == JPC - 2, b <= B - JPC - 2))
            def _():
                wait_idx()

            start_gather(b + 2, kn)
        return carry
    lax.fori_loop(0, 41, body, 0)

    wait_gather(2); add_pos(2); start_scatter(B - 3, 2)
    wait_scatter(1); start_gather(B - 1, 1)
    wait_gather(0); add_pos(0); start_scatter(B - 2, 0)
    wait_gather(1); add_pos(1); start_scatter(B - 1, 1)
    wait_scatter(2)
    wait_scatter(0)
    wait_scatter(1)


def kernel(token_indices, token_embedding, pos_embedding):
    b, h, w = token_indices.shape
    idx_t = (
        token_indices.astype(jnp.int32)
        .reshape(B, NW, PW)
        .transpose(1, 0, 2)
        .reshape(NW, B * PW)
    )
    pos_flat = pos_embedding.reshape(HW * D)
    pos2d = pos_embedding.reshape(HW, D)
    return _embed(idx_t, token_embedding, pos_flat, pos2d)


# speedup vs baseline: 2.2864x; 2.2864x over previous
import functools

import jax
import jax.numpy as jnp
from jax import lax
from jax.experimental import pallas as pl
from jax.experimental.pallas import tpu as pltpu
from jax.experimental.pallas import tpu_sc as plsc

NUM_TOKENS = 65536
D = 1024
B = 128
HW = 1024
NC = 2
NS = 16
NW = NC * NS
PW = HW // NW
LANES = 16
NBUF = 3
JPC = 4
CHUNK = JPC * PW
NTAIL = 17
POSV = PW * D - NTAIL * LANES

_mesh = plsc.VectorSubcoreMesh(core_axis_name="c", subcore_axis_name="s")


@functools.partial(
    pl.kernel,
    mesh=_mesh,
    out_type=jax.ShapeDtypeStruct((B, HW, D), jnp.float32),
    scratch_types=[
        pltpu.VMEM((2, CHUNK), jnp.int32),
        pltpu.VMEM((POSV,), jnp.float32),
        pltpu.VMEM((PW, D), jnp.float32),
        pltpu.VMEM((PW, D), jnp.float32),
        pltpu.VMEM((PW, D), jnp.float32),
        pltpu.SemaphoreType.DMA,
        pltpu.SemaphoreType.DMA,
        pltpu.SemaphoreType.DMA,
        pltpu.SemaphoreType.DMA,
        pltpu.SemaphoreType.DMA,
        pltpu.SemaphoreType.DMA,
        pltpu.SemaphoreType.DMA,
    ],
)
def _embed(idx_hbm, table_hbm, pos_hbm, pos2d_hbm, out_hbm, idxring, posv,
           r0, r1, r2, g0, g1, g2, s0, s1, s2, sidx):
    wid = lax.axis_index("s") * NC + lax.axis_index("c")
    bufs = (r0, r1, r2)
    gsems = (g0, g1, g2)
    ssems = (s0, s1, s2)

    pltpu.sync_copy(
        pos2d_hbm.at[pl.ds(wid * PW + PW - 8, 8), :],
        r0.at[pl.ds(0, 8), :],
    )
    tail_c0 = D // LANES - NTAIL
    tail = tuple(
        r0[7, pl.ds((tail_c0 + i) * LANES, LANES)] for i in range(NTAIL)
    )

    pltpu.sync_copy(pos_hbm.at[pl.ds(wid * PW * D, POSV)], posv)
    pltpu.sync_copy(idx_hbm.at[wid, pl.ds(0, CHUNK)], idxring.at[0])

    def issue_idx(c):
        pltpu.make_async_copy(
            idx_hbm.at[wid, pl.ds(c * CHUNK, CHUNK)],
            idxring.at[c % 2], sidx,
        ).start()

    def wait_idx():
        pltpu.make_async_copy(
            idx_hbm.at[wid, pl.ds(0, CHUNK)], idxring.at[0], sidx,
        ).wait()

    def start_gather(j, k):
        chunk_slot = (j // JPC) % 2
        off = (j % JPC) * PW
        pltpu.make_async_copy(
            table_hbm.at[idxring.at[chunk_slot, pl.ds(off, PW)]],
            bufs[k], gsems[k],
        ).start()

    def wait_gather(k):
        pltpu.make_async_copy(
            table_hbm.at[idxring.at[0, pl.ds(0, PW)]], bufs[k], gsems[k],
        ).wait()

    def start_scatter(b, k):
        pltpu.make_async_copy(
            bufs[k], out_hbm.at[b, pl.ds(wid * PW, PW), :], ssems[k]
        ).start()

    def wait_scatter(k):
        pltpu.make_async_copy(
            bufs[k], out_hbm.at[0, pl.ds(wid * PW, PW), :], ssems[k]
        ).wait()

    def add_pos(k):
        def add_row(r, carry):
            base = pl.multiple_of(r * D, D)

            @plsc.parallel_loop(0, D // LANES, unroll=4)
            def add_col(c):
                plsc.addupdate(
                    bufs[k].at[r, pl.ds(c * LANES, LANES)],
                    posv[pl.ds(base + c * LANES, LANES)],
                )
            return carry
        lax.fori_loop(0, PW - 1, add_row, 0)

        r31 = (PW - 1) * D

        @plsc.parallel_loop(0, D // LANES - NTAIL)
        def add_col31(c):
            plsc.addupdate(
                bufs[k].at[PW - 1, pl.ds(c * LANES, LANES)],
                posv[pl.ds(r31 + c * LANES, LANES)],
            )
        for i in range(NTAIL):
            c = D // LANES - NTAIL + i
            plsc.addupdate(bufs[k].at[PW - 1, pl.ds(c * LANES, LANES)], tail[i])

    start_gather(0, 0)
    start_gather(1, 1)
    issue_idx(1)
    wait_gather(0); add_pos(0); start_scatter(0, 0); start_gather(2, 2)
    wait_gather(1); add_pos(1); start_scatter(1, 1)
    wait_scatter(0); start_gather(3, 0)

    def body(m, carry):
        for t in range(NBUF):
            b = 3 * m + 2 + t
            k = (2 + t) % NBUF
            kn = (k + 2) % NBUF
            wait_gather(k)

            @pl.when(jnp.logical_and(b % JPC == 0, b <= B - 2 * JPC))
            def _():
                issue_idx(b // JPC + 1)

            add_pos(k)
            start_scatter(b, k)
            wait_scatter(kn)

            @pl.when(jnp.logical_and(b % JPC == JPC - 2, b <= B - JPC - 2))
            def _():
                wait_idx()

            start_gather(b + 2, kn)
        return carry
    lax.fori_loop(0, 41, body, 0)

    wait_gather(2); add_pos(2); start_scatter(B - 3, 2)
    wait_scatter(1); start_gather(B - 1, 1)
    wait_gather(0); add_pos(0); start_scatter(B - 2, 0)
    wait_gather(1); add_pos(1); start_scatter(B - 1, 1)
    wait_scatter(2)
    wait_scatter(0)
    wait_scatter(1)


def kernel(token_indices, token_embedding, pos_embedding):
    b, h, w = token_indices.shape
    idx_t = (
        token_indices.astype(jnp.int32)
        .reshape(B, NW, PW)
        .transpose(1, 0, 2)
        .reshape(NW, B * PW)
    )
    pos_flat = pos_embedding.reshape(HW * D)
    pos2d = pos_embedding.reshape(HW, D)
    return _embed(idx_t, token_embedding, pos_flat, pos2d)


# speedup vs baseline: 2.3708x; 1.0369x over previous
import functools

import jax
import jax.numpy as jnp
from jax import lax
from jax.experimental import pallas as pl
from jax.experimental.pallas import tpu as pltpu
from jax.experimental.pallas import tpu_sc as plsc

NUM_TOKENS = 65536
D = 1024
B = 128
HW = 1024
NC = 2
NS = 16
NW = NC * NS
PW = HW // NW
LANES = 16
NBUF = 3
JPC = 4
CHUNK = JPC * PW
NTAIL = 17
POSV = PW * D - NTAIL * LANES

_mesh = plsc.VectorSubcoreMesh(core_axis_name="c", subcore_axis_name="s")


@functools.partial(
    pl.kernel,
    mesh=_mesh,
    out_type=jax.ShapeDtypeStruct((B, HW, D), jnp.float32),
    scratch_types=[
        pltpu.VMEM((2, CHUNK), jnp.int32),
        pltpu.VMEM((POSV,), jnp.float32),
        pltpu.VMEM((PW, D), jnp.float32),
        pltpu.VMEM((PW, D), jnp.float32),
        pltpu.VMEM((PW, D), jnp.float32),
        pltpu.SemaphoreType.DMA,
        pltpu.SemaphoreType.DMA,
        pltpu.SemaphoreType.DMA,
        pltpu.SemaphoreType.DMA,
        pltpu.SemaphoreType.DMA,
        pltpu.SemaphoreType.DMA,
        pltpu.SemaphoreType.DMA,
    ],
)
def _embed(idx_hbm, table_hbm, pos_hbm, pos2d_hbm, out_hbm, idxring, posv,
           r0, r1, r2, g0, g1, g2, s0, s1, s2, sidx):
    wid = lax.axis_index("s") * NC + lax.axis_index("c")
    bufs = (r0, r1, r2)
    gsems = (g0, g1, g2)
    ssems = (s0, s1, s2)

    pltpu.sync_copy(
        pos2d_hbm.at[pl.ds(wid * PW + PW - 8, 8), :],
        r0.at[pl.ds(0, 8), :],
    )
    tail_c0 = D // LANES - NTAIL
    tail = tuple(
        r0[7, pl.ds((tail_c0 + i) * LANES, LANES)] for i in range(NTAIL)
    )

    pltpu.sync_copy(pos_hbm.at[pl.ds(wid * PW * D, POSV)], posv)
    pltpu.sync_copy(idx_hbm.at[wid, pl.ds(0, CHUNK)], idxring.at[0])

    def issue_idx(c):
        pltpu.make_async_copy(
            idx_hbm.at[wid, pl.ds(c * CHUNK, CHUNK)],
            idxring.at[c % 2], sidx,
        ).start()

    def wait_idx():
        pltpu.make_async_copy(
            idx_hbm.at[wid, pl.ds(0, CHUNK)], idxring.at[0], sidx,
        ).wait()

    def start_gather(j, k):
        chunk_slot = (j // JPC) % 2
        off = (j % JPC) * PW
        pltpu.make_async_copy(
            table_hbm.at[idxring.at[chunk_slot, pl.ds(off, PW)]],
            bufs[k], gsems[k],
        ).start()

    def wait_gather(k):
        pltpu.make_async_copy(
            table_hbm.at[idxring.at[0, pl.ds(0, PW)]], bufs[k], gsems[k],
        ).wait()

    def start_scatter(b, k):
        pltpu.make_async_copy(
            bufs[k], out_hbm.at[b, pl.ds(wid * PW, PW), :], ssems[k]
        ).start()

    def wait_scatter(k):
        pltpu.make_async_copy(
            bufs[k], out_hbm.at[0, pl.ds(wid * PW, PW), :], ssems[k]
        ).wait()

    def add_pos(k):
        def add_row(r, carry):
            base = pl.multiple_of(r * D, D)

            @plsc.parallel_loop(0, D // LANES, unroll=8)
            def add_col(c):
                plsc.addupdate(
                    bufs[k].at[r, pl.ds(c * LANES, LANES)],
                    posv[pl.ds(base + c * LANES, LANES)],
                )
            return carry
        lax.fori_loop(0, PW - 1, add_row, 0)

        r31 = (PW - 1) * D

        @plsc.parallel_loop(0, D // LANES - NTAIL)
        def add_col31(c):
            plsc.addupdate(
                bufs[k].at[PW - 1, pl.ds(c * LANES, LANES)],
                posv[pl.ds(r31 + c * LANES, LANES)],
            )
        for i in range(NTAIL):
            c = D // LANES - NTAIL + i
            plsc.addupdate(bufs[k].at[PW - 1, pl.ds(c * LANES, LANES)], tail[i])

    start_gather(0, 0)
    start_gather(1, 1)
    issue_idx(1)
    wait_gather(0); add_pos(0); start_scatter(0, 0); start_gather(2, 2)
    wait_gather(1); add_pos(1); start_scatter(1, 1)
    wait_scatter(0); start_gather(3, 0)

    def body(m, carry):
        for t in range(NBUF):
            b = 3 * m + 2 + t
            k = (2 + t) % NBUF
            kn = (k + 2) % NBUF
            wait_gather(k)

            @pl.when(jnp.logical_and(b % JPC == 0, b <= B - 2 * JPC))
            def _():
                issue_idx(b // JPC + 1)

            add_pos(k)
            start_scatter(b, k)
            wait_scatter(kn)

            @pl.when(jnp.logical_and(b % JPC == JPC - 2, b <= B - JPC - 2))
            def _():
                wait_idx()

            start_gather(b + 2, kn)
        return carry
    lax.fori_loop(0, 41, body, 0)

    wait_gather(2); add_pos(2); start_scatter(B - 3, 2)
    wait_scatter(1); start_gather(B - 1, 1)
    wait_gather(0); add_pos(0); start_scatter(B - 2, 0)
    wait_gather(1); add_pos(1); start_scatter(B - 1, 1)
    wait_scatter(2)
    wait_scatter(0)
    wait_scatter(1)


def kernel(token_indices, token_embedding, pos_embedding):
    b, h, w = token_indices.shape
    idx_t = (
        token_indices.astype(jnp.int32)
        .reshape(B, NW, PW)
        .transpose(1, 0, 2)
        .reshape(NW, B * PW)
    )
    pos_flat = pos_embedding.reshape(HW * D)
    pos2d = pos_embedding.reshape(HW, D)
    return _embed(idx_t, token_embedding, pos_flat, pos2d)
